# baseline (speedup 1.0000x reference)
import jax
import jax.numpy as jnp
from jax.experimental import pallas as pl
from jax.experimental.pallas import tpu as pltpu

_B, _C, _H, _W = 8, 128, 128, 128
_F = 32
_P = 2


def _gemm_kernel(idx_ref, w_ref, xa_ref, xb_ref, o_ref, w2_ref):
    @pl.when(pl.program_id(0) == 0)
    def _():
        idxv = idx_ref[0, :]
        cols = jax.lax.broadcasted_iota(jnp.int32, (_C, _C), 1)
        onehot_t = (idxv[:, None] == cols).astype(jnp.float32)
        w_eff = jnp.dot(w_ref[...], onehot_t,
                        preferred_element_type=jnp.float32)
        w_hi = w_eff.astype(jnp.bfloat16)
        zero = jnp.zeros((_F, _C), jnp.bfloat16)
        w2_ref[...] = jnp.concatenate(
            [jnp.concatenate(
                [w_hi if i == j else zero for j in range(_P)], axis=1)
             for i in range(_P)], axis=0)

    w2 = w2_ref[...]
    xb = jnp.concatenate([xa_ref[0], xb_ref[0]], axis=0)
    xt = jnp.swapaxes(xb.astype(jnp.bfloat16), 0, 1)
    xr = xt.reshape(_H * _C, _W)
    outs = []
    for h2 in range(_H // _P):
        seg = xr[h2 * _P * _C:(h2 + 1) * _P * _C, :]
        outs.append(jnp.dot(w2, seg, preferred_element_type=jnp.float32))
    ot = jnp.concatenate(outs, axis=0).reshape(_H, _F, _W)
    o_ref[0] = jnp.swapaxes(ot, 0, 1)


def kernel(x, weights, channel_idx):
    w2 = weights.reshape(_F, _C)
    idx2 = channel_idx.reshape(1, _C)
    out = pl.pallas_call(
        _gemm_kernel,
        grid=(_B,),
        in_specs=[
            pl.BlockSpec((1, _C), lambda b: (0, 0)),
            pl.BlockSpec((_F, _C), lambda b: (0, 0)),
            pl.BlockSpec((1, _C // 2, _H, _W), lambda b: (b, 0, 0, 0)),
            pl.BlockSpec((1, _C // 2, _H, _W), lambda b: (b, 1, 0, 0)),
        ],
        out_specs=pl.BlockSpec((1, _F, _H, _W), lambda b: (b, 0, 0, 0)),
        out_shape=jax.ShapeDtypeStruct((_B, _F, _H, _W), jnp.float32),
        scratch_shapes=[pltpu.VMEM((_P * _F, _P * _C), jnp.bfloat16)],
    )(idx2, w2, x, x)
    return out


# speedup vs baseline: 1.0345x; 1.0345x over previous
import jax
import jax.numpy as jnp
from jax.experimental import pallas as pl
from jax.experimental.pallas import tpu as pltpu

_B, _C, _H, _W = 8, 128, 128, 128
_F = 32
_P = 2


def _gemm_kernel(idx_ref, w_ref, x_ref, o_ref, w2_ref):
    @pl.when(pl.program_id(0) == 0)
    def _():
        idxv = idx_ref[0, :]
        cols = jax.lax.broadcasted_iota(jnp.int32, (_C, _C), 1)
        onehot_t = (idxv[:, None] == cols).astype(jnp.float32)
        w_eff = jnp.dot(w_ref[...], onehot_t,
                        preferred_element_type=jnp.float32)
        w_hi = w_eff.astype(jnp.bfloat16)
        zero = jnp.zeros((_F, _C), jnp.bfloat16)
        w2_ref[...] = jnp.concatenate(
            [jnp.concatenate(
                [w_hi if i == j else zero for j in range(_P)], axis=1)
             for i in range(_P)], axis=0)

    w2 = w2_ref[...]
    xt = jnp.swapaxes(x_ref[0].astype(jnp.bfloat16), 0, 1)
    xr = xt.reshape(_H * _C, _W)
    outs = []
    for h2 in range(_H // _P):
        seg = xr[h2 * _P * _C:(h2 + 1) * _P * _C, :]
        outs.append(jnp.dot(w2, seg, preferred_element_type=jnp.float32))
    ot = jnp.concatenate(outs, axis=0).reshape(_H, _F, _W)
    o_ref[0] = jnp.swapaxes(ot, 0, 1)


def kernel(x, weights, channel_idx):
    w2 = weights.reshape(_F, _C)
    idx2 = channel_idx.reshape(1, _C)
    out = pl.pallas_call(
        _gemm_kernel,
        grid=(_B,),
        in_specs=[
            pl.BlockSpec((1, _C), lambda b: (0, 0)),
            pl.BlockSpec((_F, _C), lambda b: (0, 0)),
            pl.BlockSpec((1, _C, _H, _W), lambda b: (b, 0, 0, 0)),
        ],
        out_specs=pl.BlockSpec((1, _F, _H, _W), lambda b: (b, 0, 0, 0)),
        out_shape=jax.ShapeDtypeStruct((_B, _F, _H, _W), jnp.float32),
        scratch_shapes=[pltpu.VMEM((_P * _F, _P * _C), jnp.bfloat16)],
    )(idx2, w2, x)
    return out
